# Initial kernel scaffold; baseline (speedup 1.0000x reference)
import jax
import jax.numpy as jnp
from jax.experimental import pallas as pl


def kernel(x, edge_index, W, b):
    raise NotImplementedError("write your pallas kernel here")


# speedup vs baseline: 6.4237x; 6.4237x over previous
import functools

import jax
import jax.numpy as jnp
from jax import lax
from jax.experimental import pallas as pl
from jax.experimental.pallas import tpu as pltpu
from jax.experimental.pallas import tpu_sc as plsc

N_NODES = 10000
N_PAD = 10240
H = 128
K_HOPS = 10
ALPHA = 0.1
NTILES = 32
CHUNK = 128
NCHUNK = 79
E_PAD = NTILES * NCHUNK * CHUNK
ROWS_PER_TILE = N_PAD // 16
DUMMY_DST = N_NODES + 100
BLK = 2048

_mesh = plsc.VectorSubcoreMesh(core_axis_name="c", subcore_axis_name="s")


def _deg_body(dst3, pd, idxc_v, ones_v, zer_v, zidx_v, rdbuf_v, deg_sh):
    c = lax.axis_index("c")
    s = lax.axis_index("s")
    wid = s * 2 + c
    nq = ROWS_PER_TILE // CHUNK

    def fill_ones(r, carry):
        ones_v[r, :] = jnp.ones((16,), jnp.float32)
        return carry

    lax.fori_loop(0, CHUNK, fill_ones, 0)

    def fill_zer(r, carry):
        zer_v[r, :] = jnp.zeros((16,), jnp.float32)
        return carry

    lax.fori_loop(0, CHUNK, fill_zer, 0)
    base = s * ROWS_PER_TILE
    for q in range(nq):
        for l in range(CHUNK // 16):
            zidx_v[pl.ds(l * 16, 16)] = (
                base + q * CHUNK + l * 16 + lax.iota(jnp.int32, 16))
        pltpu.sync_copy(zer_v, deg_sh.at[zidx_v])
    plsc.subcore_barrier()

    def body(j, carry):
        pltpu.sync_copy(dst3.at[wid * NCHUNK + j], idxc_v)
        pltpu.sync_copy(ones_v, deg_sh.at[idxc_v], add=True)
        return carry

    lax.fori_loop(0, NCHUNK, body, 0)
    plsc.subcore_barrier()
    for q in range(nq):
        for l in range(CHUNK // 16):
            zidx_v[pl.ds(l * 16, 16)] = (
                base + q * CHUNK + l * 16 + lax.iota(jnp.int32, 16))
        pltpu.sync_copy(deg_sh.at[zidx_v], rdbuf_v)
        pltpu.sync_copy(rdbuf_v, pd.at[wid * nq + q])


_deg_call = pl.kernel(
    _deg_body,
    out_type=jax.ShapeDtypeStruct(
        (NTILES * (ROWS_PER_TILE // CHUNK), CHUNK, 16), jnp.float32),
    mesh=_mesh,
    scratch_types=[
        pltpu.VMEM((CHUNK,), jnp.int32),
        pltpu.VMEM((CHUNK, 16), jnp.float32),
        pltpu.VMEM((CHUNK, 16), jnp.float32),
        pltpu.VMEM((CHUNK,), jnp.int32),
        pltpu.VMEM((CHUNK, 16), jnp.float32),
        pltpu.VMEM_SHARED((N_PAD, 16), jnp.float32),
    ],
)


def _scat_body(src3, dst3, u, out, sidxc_v, didxc_v, buf, zbuf, zidx_v, acc,
               sem):
    c = lax.axis_index("c")
    s = lax.axis_index("s")
    wid = s * 2 + c
    nq = ROWS_PER_TILE // CHUNK

    def zrow(r, carry):
        def zcol(l, carry2):
            zbuf[r, pl.ds(l * 16, 16)] = jnp.zeros((16,), jnp.float32)
            return carry2

        return lax.fori_loop(0, 8, zcol, carry)

    lax.fori_loop(0, CHUNK, zrow, 0)
    base = s * ROWS_PER_TILE
    for q in range(nq):
        for l in range(CHUNK // 16):
            zidx_v[pl.ds(l * 16, 16)] = (
                base + q * CHUNK + l * 16 + lax.iota(jnp.int32, 16))
        pltpu.sync_copy(zbuf, acc.at[zidx_v])
    plsc.subcore_barrier()

    def body(j, carry):
        pltpu.sync_copy(src3.at[wid * NCHUNK + j], sidxc_v)
        pltpu.sync_copy(dst3.at[wid * NCHUNK + j], didxc_v)
        pltpu.async_copy(u.at[sidxc_v], buf, sem).wait()
        pltpu.sync_copy(buf, acc.at[didxc_v], add=True)
        return carry

    lax.fori_loop(0, NCHUNK, body, 0)
    plsc.subcore_barrier()
    for q in range(nq):
        for l in range(CHUNK // 16):
            zidx_v[pl.ds(l * 16, 16)] = (
                base + q * CHUNK + l * 16 + lax.iota(jnp.int32, 16))
        pltpu.sync_copy(acc.at[zidx_v], zbuf)
        pltpu.sync_copy(zbuf, out.at[wid * nq + q])


_scat_call = pl.kernel(
    _scat_body,
    out_type=jax.ShapeDtypeStruct(
        (NTILES * (ROWS_PER_TILE // CHUNK), CHUNK, H), jnp.float32),
    mesh=_mesh,
    scratch_types=[
        pltpu.VMEM((CHUNK,), jnp.int32),
        pltpu.VMEM((CHUNK,), jnp.int32),
        pltpu.VMEM((CHUNK, H), jnp.float32),
        pltpu.VMEM((CHUNK, H), jnp.float32),
        pltpu.VMEM((CHUNK,), jnp.int32),
        pltpu.VMEM_SHARED((N_PAD, H), jnp.float32),
        pltpu.SemaphoreType.DMA,
    ],
)


def _prep_kernel(x_ref, wt_ref, b_ref, pd0_ref, pd1_ref, u0_ref):
    h = jnp.dot(x_ref[...], wt_ref[...], preferred_element_type=jnp.float32)
    h = h + b_ref[...]
    deg = pd0_ref[0, :, 0:1] + pd1_ref[0, :, 0:1] + 1.0
    u0_ref[...] = lax.rsqrt(deg) * h


_prep_call = pl.pallas_call(
    _prep_kernel,
    grid=(N_PAD // BLK,),
    in_specs=[
        pl.BlockSpec((BLK, H), lambda i: (i, 0)),
        pl.BlockSpec((H, H), lambda i: (0, 0)),
        pl.BlockSpec((1, H), lambda i: (0, 0)),
        pl.BlockSpec((1, BLK, 16), lambda i: (0, i, 0)),
        pl.BlockSpec((1, BLK, 16), lambda i: (1, i, 0)),
    ],
    out_specs=pl.BlockSpec((BLK, H), lambda i: (i, 0)),
    out_shape=jax.ShapeDtypeStruct((N_PAD, H), jnp.float32),
)


def _upd_kernel(pa_ref, pb_ref, u_ref, u0_ref, pd0_ref, pd1_ref, o_ref, *, final):
    deg = pd0_ref[0, :, 0:1] + pd1_ref[0, :, 0:1] + 1.0
    ssum = pa_ref[0] + pb_ref[0] + u_ref[...]
    un = (1.0 - ALPHA) * ssum / deg + ALPHA * u0_ref[...]
    if final:
        un = un * jnp.sqrt(deg)
    o_ref[...] = un


def _make_upd(final):
    return pl.pallas_call(
        functools.partial(_upd_kernel, final=final),
        grid=(N_PAD // BLK,),
        in_specs=[
            pl.BlockSpec((1, BLK, H), lambda i: (0, i, 0)),
            pl.BlockSpec((1, BLK, H), lambda i: (1, i, 0)),
            pl.BlockSpec((BLK, H), lambda i: (i, 0)),
            pl.BlockSpec((BLK, H), lambda i: (i, 0)),
            pl.BlockSpec((1, BLK, 16), lambda i: (0, i, 0)),
            pl.BlockSpec((1, BLK, 16), lambda i: (1, i, 0)),
        ],
        out_specs=pl.BlockSpec((BLK, H), lambda i: (i, 0)),
        out_shape=jax.ShapeDtypeStruct((N_PAD, H), jnp.float32),
    )


_upd_call = _make_upd(False)
_upd_final_call = _make_upd(True)


def kernel(x, edge_index, W, b):
    src = edge_index[0].astype(jnp.int32)
    dst = edge_index[1].astype(jnp.int32)
    e = src.shape[0]
    src3 = jnp.zeros((E_PAD,), jnp.int32).at[:e].set(src).reshape(
        NTILES * NCHUNK, CHUNK)
    dst3 = jnp.full((E_PAD,), DUMMY_DST, jnp.int32).at[:e].set(dst).reshape(
        NTILES * NCHUNK, CHUNK)
    xp = jnp.pad(x, ((0, N_PAD - x.shape[0]), (0, 0)))
    wt = W.T
    b1 = b.reshape(1, H)

    pd160 = _deg_call(dst3)
    pd = pd160.reshape(16, 2, ROWS_PER_TILE, 16).transpose(1, 0, 2, 3).reshape(
        2, N_PAD, 16)
    u = _prep_call(xp, wt, b1, pd, pd)
    u0 = u
    for k in range(K_HOPS):
        p160 = _scat_call(src3, dst3, u)
        partial = p160.reshape(16, 2, ROWS_PER_TILE, H).transpose(
            1, 0, 2, 3).reshape(2, N_PAD, H)
        upd = _upd_final_call if k == K_HOPS - 1 else _upd_call
        u = upd(partial, partial, u, u0, pd, pd)
    return u[:N_NODES]
